# baseline (speedup 1.0000x reference)
import jax
import jax.numpy as jnp
from jax.experimental import pallas as pl
from jax.experimental.pallas import tpu as pltpu

B, L, D = 8, 2048, 1024
TL = 2048
SB = 128
EPS = 1e-4


def _hnet_kernel(x_ref, w_ref, bres_ref, out_ref, zprev_ref, qprev_ref):
    l = pl.program_id(1)
    first = l == 0

    x_blk = x_ref[0]

    qv = jax.lax.dot_general(x_blk, w_ref[:, :D], (((1,), (0,)), ((), ())),
                             preferred_element_type=jnp.float32)
    kv = jax.lax.dot_general(x_blk, w_ref[:, D:2 * D], (((1,), (0,)), ((), ())),
                             preferred_element_type=jnp.float32)
    res = jax.lax.dot_general(x_blk, w_ref[:, 2 * D:], (((1,), (0,)), ((), ())),
                              preferred_element_type=jnp.float32)

    qprev_row = qprev_ref[...]
    qs = jnp.concatenate([qprev_row, qv[:-1]], axis=0)
    qprev_ref[...] = qv[-1:]

    qk = jnp.sum(qs * kv, axis=1, keepdims=True)
    nq = jnp.sum(qv * qv, axis=1, keepdims=True)
    nq_prev = jnp.sum(qprev_row * qprev_row, axis=1, keepdims=True)
    qq = jnp.concatenate([nq_prev, nq[:-1]], axis=0)
    kk = jnp.sum(kv * kv, axis=1, keepdims=True)
    denom = jnp.maximum(jnp.sqrt(qq), 1e-8) * jnp.maximum(jnp.sqrt(kk), 1e-8)
    cos = qk / denom
    p_raw = jnp.clip(0.5 - 0.5 * cos, 0.0, 1.0)

    row = jax.lax.broadcasted_iota(jnp.int32, (TL, 1), 0)
    p_raw = jnp.where(first & (row == 0), 1.0, p_raw)

    bsel = p_raw >= 0.5
    p_eff = jnp.where(bsel, jnp.clip(p_raw, EPS, 1.0 - EPS), 0.0)
    a = 1.0 - p_eff
    la = jnp.log(a)

    rows = jax.lax.broadcasted_iota(jnp.int32, (SB, SB), 0)
    cols = jax.lax.broadcasted_iota(jnp.int32, (SB, SB), 1)
    lower = rows >= cols
    ones_tri = jnp.where(lower, 1.0, 0.0)

    @pl.when(first)
    def _():
        zprev_ref[...] = jnp.zeros_like(zprev_ref)

    carry = zprev_ref[...]
    css, zs = [], []
    for i in range(TL // SB):
        sl = slice(i * SB, (i + 1) * SB)
        la_s = la[sl]
        cs = jax.lax.dot_general(ones_tri, la_s, (((1,), (0,)), ((), ())),
                                 precision=jax.lax.Precision.HIGHEST,
                                 preferred_element_type=jnp.float32)
        T = jnp.where(lower, jnp.exp(cs - cs.T), 0.0)
        bv = p_eff[sl] * x_blk[sl]
        z = jax.lax.dot_general(T, bv, (((1,), (0,)), ((), ())),
                                preferred_element_type=jnp.float32)
        css.append(jnp.exp(cs))
        zs.append(z)
    for i in range(TL // SB):
        sl = slice(i * SB, (i + 1) * SB)
        z = zs[i] + css[i] * carry
        carry = z[-1:]
        out_ref[0, sl, :] = res[sl] + bres_ref[...] + z
    zprev_ref[...] = carry


@jax.jit
def kernel(x, Wq, Wk, Wres, bres):
    w_all = jnp.concatenate([Wq, Wk, Wres], axis=1)
    bres2d = bres.reshape(1, D)
    grid = (B, L // TL)
    return pl.pallas_call(
        _hnet_kernel,
        grid=grid,
        compiler_params=pltpu.CompilerParams(
            dimension_semantics=("parallel", "arbitrary"),
            vmem_limit_bytes=120 * 1024 * 1024),
        in_specs=[
            pl.BlockSpec((1, TL, D), lambda b, l: (b, l, 0)),
            pl.BlockSpec((D, 3 * D), lambda b, l: (0, 0)),
            pl.BlockSpec((1, D), lambda b, l: (0, 0)),
        ],
        out_specs=pl.BlockSpec((1, TL, D), lambda b, l: (b, l, 0)),
        out_shape=jax.ShapeDtypeStruct((B, L, D), jnp.float32),
        scratch_shapes=[
            pltpu.VMEM((1, D), jnp.float32),
            pltpu.VMEM((1, D), jnp.float32),
        ],
    )(x, w_all, bres2d)


# speedup vs baseline: 1.0226x; 1.0226x over previous
import jax
import jax.numpy as jnp
from jax.experimental import pallas as pl
from jax.experimental.pallas import tpu as pltpu

B, L, D = 8, 2048, 1024
TL = 1024
SB = 128
EPS = 1e-4


def _hnet_kernel(x_ref, w_ref, bres_ref, out_ref, zprev_ref, qprev_ref):
    l = pl.program_id(1)
    first = l == 0

    x_blk = x_ref[0]

    qv = jax.lax.dot_general(x_blk, w_ref[:, :D], (((1,), (0,)), ((), ())),
                             preferred_element_type=jnp.float32)
    kv = jax.lax.dot_general(x_blk, w_ref[:, D:2 * D], (((1,), (0,)), ((), ())),
                             preferred_element_type=jnp.float32)
    res = jax.lax.dot_general(x_blk, w_ref[:, 2 * D:], (((1,), (0,)), ((), ())),
                              preferred_element_type=jnp.float32)

    qprev_row = qprev_ref[...]
    qs = jnp.concatenate([qprev_row, qv[:-1]], axis=0)
    qprev_ref[...] = qv[-1:]

    qk = jnp.sum(qs * kv, axis=1, keepdims=True)
    nq = jnp.sum(qv * qv, axis=1, keepdims=True)
    nq_prev = jnp.sum(qprev_row * qprev_row, axis=1, keepdims=True)
    qq = jnp.concatenate([nq_prev, nq[:-1]], axis=0)
    kk = jnp.sum(kv * kv, axis=1, keepdims=True)
    denom = jnp.maximum(jnp.sqrt(qq), 1e-8) * jnp.maximum(jnp.sqrt(kk), 1e-8)
    cos = qk / denom
    p_raw = jnp.clip(0.5 - 0.5 * cos, 0.0, 1.0)

    row = jax.lax.broadcasted_iota(jnp.int32, (TL, 1), 0)
    p_raw = jnp.where(first & (row == 0), 1.0, p_raw)

    bsel = p_raw >= 0.5
    p_eff = jnp.where(bsel, jnp.clip(p_raw, EPS, 1.0 - EPS), 0.0)
    a = 1.0 - p_eff
    la = jnp.log(a)

    rows = jax.lax.broadcasted_iota(jnp.int32, (SB, SB), 0)
    cols = jax.lax.broadcasted_iota(jnp.int32, (SB, SB), 1)
    lower = rows >= cols
    ones_tri = jnp.where(lower, 1.0, 0.0)

    @pl.when(first)
    def _():
        zprev_ref[...] = jnp.zeros_like(zprev_ref)

    carry = zprev_ref[...]
    css, zs = [], []
    for i in range(TL // SB):
        sl = slice(i * SB, (i + 1) * SB)
        la_s = la[sl]
        cs = jax.lax.dot_general(ones_tri, la_s, (((1,), (0,)), ((), ())),
                                 precision=jax.lax.Precision.HIGHEST,
                                 preferred_element_type=jnp.float32)
        T = jnp.where(lower, jnp.exp(cs - cs.T), 0.0)
        bv = p_eff[sl] * x_blk[sl]
        z = jax.lax.dot_general(T, bv, (((1,), (0,)), ((), ())),
                                preferred_element_type=jnp.float32)
        css.append(jnp.exp(cs))
        zs.append(z)
    for i in range(TL // SB):
        sl = slice(i * SB, (i + 1) * SB)
        z = zs[i] + css[i] * carry
        carry = z[-1:]
        out_ref[0, sl, :] = res[sl] + bres_ref[...] + z
    zprev_ref[...] = carry


@jax.jit
def kernel(x, Wq, Wk, Wres, bres):
    w_all = jnp.concatenate([Wq, Wk, Wres], axis=1)
    bres2d = bres.reshape(1, D)
    grid = (B, L // TL)
    return pl.pallas_call(
        _hnet_kernel,
        grid=grid,
        compiler_params=pltpu.CompilerParams(
            dimension_semantics=("parallel", "arbitrary"),
            vmem_limit_bytes=120 * 1024 * 1024),
        in_specs=[
            pl.BlockSpec((1, TL, D), lambda b, l: (b, l, 0)),
            pl.BlockSpec((D, 3 * D), lambda b, l: (0, 0)),
            pl.BlockSpec((1, D), lambda b, l: (0, 0)),
        ],
        out_specs=pl.BlockSpec((1, TL, D), lambda b, l: (b, l, 0)),
        out_shape=jax.ShapeDtypeStruct((B, L, D), jnp.float32),
        scratch_shapes=[
            pltpu.VMEM((1, D), jnp.float32),
            pltpu.VMEM((1, D), jnp.float32),
        ],
    )(x, w_all, bres2d)
